# baseline (speedup 1.0000x reference)
import functools

import jax
import jax.numpy as jnp
from jax import lax
from jax.experimental import pallas as pl
from jax.experimental.pallas import tpu as pltpu
from jax.experimental.pallas import tpu_sc as plsc

BLK = 512
CHUNK = 128
HW = 8


def _dist_argmin_kernel(z_ref, wt_ref, idx_ref, loss_ref, w2_s):
    i = pl.program_id(0)
    blk, k = z_ref.shape[0], wt_ref.shape[1]

    @pl.when(i == 0)
    def _init():
        wt0 = wt_ref[...]
        w2_s[...] = jnp.sum(wt0 * wt0, axis=0, keepdims=True) * 0.25
        loss_ref[...] = jnp.zeros_like(loss_ref)

    z = z_ref[...]
    z2 = jnp.sum(z * z, axis=1, keepdims=True)
    scores2 = jnp.dot(z, wt_ref[...], preferred_element_type=jnp.float32)
    dist = (z2 + w2_s[...]) - scores2
    m = jnp.min(dist, axis=1, keepdims=True)
    lanes = lax.broadcasted_iota(jnp.int32, (1, k), 1).astype(jnp.float32)
    idx_f = jnp.min(jnp.where(dist == m, lanes, float(k)), axis=1, keepdims=True)
    idx_ref[...] = idx_f.astype(jnp.int32).reshape(1, blk // 128, 128)
    loss_ref[...] = loss_ref[...] + jnp.full((1, 1), jnp.sum(m), jnp.float32)


def _make_sc_gather_hist(n_tok, n_emb, dim, nc, ns):
    nw = nc * ns
    bpw = n_tok // nw
    nch = bpw // CHUNK
    half = nch // 2

    mesh = plsc.VectorSubcoreMesh(core_axis_name="c", subcore_axis_name="s")

    @functools.partial(
        pl.kernel,
        out_type=[
            jax.ShapeDtypeStruct((n_tok, dim), jnp.float32),
            jax.ShapeDtypeStruct((nc, n_emb, HW), jnp.float32),
        ],
        mesh=mesh,
        scratch_types=[
            pltpu.VMEM((nch, CHUNK), jnp.int32),
            pltpu.VMEM((half * CHUNK, dim), jnp.float32),
            pltpu.VMEM((CHUNK, HW), jnp.float32),
            pltpu.VMEM_SHARED((n_emb, HW), jnp.float32),
            pltpu.SemaphoreType.DMA,
            pltpu.SemaphoreType.DMA,
        ],
        compiler_params=pltpu.CompilerParams(use_tc_tiling_on_sc=False),
    )
    def sc_kernel(w_hbm, idx_hbm, ones_hbm, zeros_hbm, q_hbm, cnt_hbm,
                  idx_v, rows_v, ones_v, counts_v, sem_g, sem_h):
        c = lax.axis_index("c")
        s = lax.axis_index("s")
        wid = s * nc + c
        rps = n_emb // ns
        pltpu.sync_copy(idx_hbm.at[pl.ds(wid * nch, nch)], idx_v)
        pltpu.sync_copy(zeros_hbm.at[pl.ds(s * rps, rps)],
                        counts_v.at[pl.ds(s * rps, rps)])
        pltpu.sync_copy(ones_hbm, ones_v)
        plsc.subcore_barrier()
        hists = [
            pltpu.async_copy(ones_v, counts_v.at[idx_v.at[j]], sem_h, add=True)
            for j in range(nch)
        ]
        for h2 in range(2):
            gathers = [
                pltpu.async_copy(w_hbm.at[idx_v.at[h2 * half + j]],
                                 rows_v.at[pl.ds(j * CHUNK, CHUNK)], sem_g)
                for j in range(half)
            ]
            for g in gathers:
                g.wait()
            pltpu.sync_copy(
                rows_v, q_hbm.at[pl.ds(wid * bpw + h2 * half * CHUNK,
                                       half * CHUNK)])
        for h in hists:
            h.wait()
        plsc.subcore_barrier()

        @pl.when(s == 0)
        def _export():
            pltpu.sync_copy(counts_v, cnt_hbm.at[c])

    return sc_kernel


def _finalize_kernel(cnt_ref, loss_ref, loss_out, perp_out, *, n_tok, dim):
    counts = jnp.sum(cnt_ref[...], axis=0, keepdims=True)
    probs = counts * (1.0 / n_tok)
    ent = -jnp.sum(probs * jnp.log(probs + 1e-10))
    perp_out[...] = jnp.full((1, 1), jnp.exp(ent), dtype=jnp.float32)
    loss_out[...] = jnp.full(
        (1, 1), 0.25 * loss_ref[0, 0] / (n_tok * dim), dtype=jnp.float32)


def _vq_forward(z_e, w, blk, interpret=False):
    n_tok, dim = z_e.shape
    n_emb = w.shape[0]
    nb = n_tok // blk
    idx, loss_sum = pl.pallas_call(
        _dist_argmin_kernel,
        grid=(nb,),
        in_specs=[
            pl.BlockSpec((blk, dim), lambda i: (i, 0)),
            pl.BlockSpec((dim, n_emb), lambda i: (0, 0)),
        ],
        out_specs=[
            pl.BlockSpec((1, blk // 128, 128), lambda i: (i, 0, 0)),
            pl.BlockSpec((1, 1), lambda i: (0, 0)),
        ],
        out_shape=[
            jax.ShapeDtypeStruct((nb, blk // 128, 128), jnp.int32),
            jax.ShapeDtypeStruct((1, 1), jnp.float32),
        ],
        scratch_shapes=[pltpu.VMEM((1, n_emb), jnp.float32)],
        compiler_params=pltpu.CompilerParams(
            dimension_semantics=("arbitrary",),
        ),
        interpret=interpret,
    )(z_e, 2.0 * w.T)

    info = plsc.get_sparse_core_info()
    nc, ns = info.num_cores, info.num_subcores
    sc = _make_sc_gather_hist(n_tok, n_emb, dim, nc, ns)
    idx_grid = idx.reshape(n_tok // CHUNK, CHUNK)
    ones = jnp.zeros((CHUNK, HW), jnp.float32).at[:, 0].set(1.0)
    zeros = jnp.zeros((n_emb, HW), jnp.float32)
    q, cnt = sc(w, idx_grid, ones, zeros)
    cnt = cnt[:, :, 0]

    loss, perp = pl.pallas_call(
        functools.partial(_finalize_kernel, n_tok=n_tok, dim=dim),
        out_shape=[
            jax.ShapeDtypeStruct((1, 1), jnp.float32),
            jax.ShapeDtypeStruct((1, 1), jnp.float32),
        ],
        interpret=interpret,
    )(cnt, loss_sum)
    return loss.reshape(()), q, perp.reshape(())


def kernel(z_e, embedding_weight):
    loss, q, perp = _vq_forward(z_e, embedding_weight, BLK)
    return (loss, q, perp)


# speedup vs baseline: 1.2442x; 1.2442x over previous
import functools

import jax
import jax.numpy as jnp
from jax import lax
from jax.experimental import pallas as pl
from jax.experimental.pallas import tpu as pltpu
from jax.experimental.pallas import tpu_sc as plsc

BLK = 512
CHUNK = 128
HW = 8


def _dist_argmin_kernel(z_ref, wt_ref, idx_ref, loss_ref, w2_s):
    i = pl.program_id(0)
    blk, k = z_ref.shape[0], wt_ref.shape[1]

    @pl.when(i == 0)
    def _init():
        wt0 = wt_ref[...]
        w2_s[...] = jnp.sum(wt0 * wt0, axis=0, keepdims=True)
        loss_ref[...] = jnp.zeros_like(loss_ref)

    z = z_ref[...]
    z2 = jnp.sum(z * z, axis=1, keepdims=True)
    scores = jnp.dot(z, wt_ref[...], preferred_element_type=jnp.float32)
    w2 = w2_s[...]
    lanes = lax.broadcasted_iota(jnp.int32, (1, 128), 1).astype(jnp.float32)
    v = (z2 + w2[:, 0:128]) - 2.0 * scores[:, 0:128]
    bi = jnp.broadcast_to(lanes, v.shape)
    for c in range(1, k // 128):
        sl = slice(c * 128, (c + 1) * 128)
        d = (z2 + w2[:, sl]) - 2.0 * scores[:, sl]
        cmp = d < v
        v = jnp.where(cmp, d, v)
        bi = jnp.where(cmp, lanes + float(c * 128), bi)
    m = jnp.min(v, axis=1, keepdims=True)
    idx_f = jnp.min(jnp.where(v == m, bi, float(k)), axis=1, keepdims=True)
    idx_ref[...] = idx_f.astype(jnp.int32).reshape(1, blk // 128, 128)
    loss_ref[...] = loss_ref[...] + jnp.full((1, 1), jnp.sum(m), jnp.float32)


def _make_sc_gather_hist(n_tok, n_emb, dim, nc, ns):
    nw = nc * ns
    bpw = n_tok // nw
    nch = bpw // CHUNK
    half = nch // 2

    mesh = plsc.VectorSubcoreMesh(core_axis_name="c", subcore_axis_name="s")

    @functools.partial(
        pl.kernel,
        out_type=[
            jax.ShapeDtypeStruct((n_tok, dim), jnp.float32),
            jax.ShapeDtypeStruct((nc, n_emb, HW), jnp.float32),
        ],
        mesh=mesh,
        scratch_types=[
            pltpu.VMEM((nch, CHUNK), jnp.int32),
            pltpu.VMEM((half * CHUNK, dim), jnp.float32),
            pltpu.VMEM((CHUNK, HW), jnp.float32),
            pltpu.VMEM_SHARED((n_emb, HW), jnp.float32),
            pltpu.SemaphoreType.DMA,
            pltpu.SemaphoreType.DMA,
        ],
        compiler_params=pltpu.CompilerParams(use_tc_tiling_on_sc=False),
    )
    def sc_kernel(w_hbm, idx_hbm, ones_hbm, zeros_hbm, q_hbm, cnt_hbm,
                  idx_v, rows_v, ones_v, counts_v, sem_g, sem_h):
        c = lax.axis_index("c")
        s = lax.axis_index("s")
        wid = s * nc + c
        rps = n_emb // ns
        pltpu.sync_copy(idx_hbm.at[pl.ds(wid * nch, nch)], idx_v)
        pltpu.sync_copy(zeros_hbm.at[pl.ds(s * rps, rps)],
                        counts_v.at[pl.ds(s * rps, rps)])
        pltpu.sync_copy(ones_hbm, ones_v)
        plsc.subcore_barrier()
        hists = [
            pltpu.async_copy(ones_v, counts_v.at[idx_v.at[j]], sem_h, add=True)
            for j in range(nch)
        ]
        for h2 in range(2):
            gathers = [
                pltpu.async_copy(w_hbm.at[idx_v.at[h2 * half + j]],
                                 rows_v.at[pl.ds(j * CHUNK, CHUNK)], sem_g)
                for j in range(half)
            ]
            for g in gathers:
                g.wait()
            pltpu.sync_copy(
                rows_v, q_hbm.at[pl.ds(wid * bpw + h2 * half * CHUNK,
                                       half * CHUNK)])
        for h in hists:
            h.wait()
        plsc.subcore_barrier()

        @pl.when(s == 0)
        def _export():
            pltpu.sync_copy(counts_v, cnt_hbm.at[c])

    return sc_kernel


def _finalize_kernel(cnt_ref, loss_ref, loss_out, perp_out, *, n_tok, dim):
    counts = jnp.sum(cnt_ref[...], axis=0, keepdims=True)
    probs = counts * (1.0 / n_tok)
    ent = -jnp.sum(probs * jnp.log(probs + 1e-10))
    perp_out[...] = jnp.full((1, 1), jnp.exp(ent), dtype=jnp.float32)
    loss_out[...] = jnp.full(
        (1, 1), 0.25 * loss_ref[0, 0] / (n_tok * dim), dtype=jnp.float32)


def _vq_forward(z_e, w, blk, interpret=False):
    n_tok, dim = z_e.shape
    n_emb = w.shape[0]
    nb = n_tok // blk
    idx, loss_sum = pl.pallas_call(
        _dist_argmin_kernel,
        grid=(nb,),
        in_specs=[
            pl.BlockSpec((blk, dim), lambda i: (i, 0)),
            pl.BlockSpec((dim, n_emb), lambda i: (0, 0)),
        ],
        out_specs=[
            pl.BlockSpec((1, blk // 128, 128), lambda i: (i, 0, 0)),
            pl.BlockSpec((1, 1), lambda i: (0, 0)),
        ],
        out_shape=[
            jax.ShapeDtypeStruct((nb, blk // 128, 128), jnp.int32),
            jax.ShapeDtypeStruct((1, 1), jnp.float32),
        ],
        scratch_shapes=[pltpu.VMEM((1, n_emb), jnp.float32)],
        compiler_params=pltpu.CompilerParams(
            dimension_semantics=("arbitrary",),
        ),
        interpret=interpret,
    )(z_e, w.T)

    info = plsc.get_sparse_core_info()
    nc, ns = info.num_cores, info.num_subcores
    sc = _make_sc_gather_hist(n_tok, n_emb, dim, nc, ns)
    idx_grid = idx.reshape(n_tok // CHUNK, CHUNK)
    ones = jnp.zeros((CHUNK, HW), jnp.float32).at[:, 0].set(1.0)
    zeros = jnp.zeros((n_emb, HW), jnp.float32)
    q, cnt = sc(w, idx_grid, ones, zeros)
    cnt = cnt[:, :, 0]

    loss, perp = pl.pallas_call(
        functools.partial(_finalize_kernel, n_tok=n_tok, dim=dim),
        out_shape=[
            jax.ShapeDtypeStruct((1, 1), jnp.float32),
            jax.ShapeDtypeStruct((1, 1), jnp.float32),
        ],
        interpret=interpret,
    )(cnt, loss_sum)
    return loss.reshape(()), q, perp.reshape(())


def kernel(z_e, embedding_weight):
    loss, q, perp = _vq_forward(z_e, embedding_weight, BLK)
    return (loss, q, perp)


# speedup vs baseline: 1.3642x; 1.0964x over previous
import functools

import jax
import jax.numpy as jnp
from jax import lax
from jax.experimental import pallas as pl
from jax.experimental.pallas import tpu as pltpu
from jax.experimental.pallas import tpu_sc as plsc

BLK = 512
CHUNK = 128
HW = 8


def _dist_argmin_kernel(z_ref, wt_ref, idx_ref, loss_ref, w2_s):
    i = pl.program_id(0)
    blk, k = z_ref.shape[0], wt_ref.shape[1]

    @pl.when(i == 0)
    def _init():
        wt0 = wt_ref[...]
        w2_s[...] = jnp.sum(wt0 * wt0, axis=0, keepdims=True)
        loss_ref[...] = jnp.zeros_like(loss_ref)

    z = z_ref[...]
    z2 = jnp.sum(z * z, axis=1, keepdims=True) * 0.25
    scores = jnp.dot(z, wt_ref[...], preferred_element_type=jnp.float32)
    w2 = w2_s[...]
    lanes = lax.broadcasted_iota(jnp.int32, (1, 128), 1).astype(jnp.float32)
    v = (z2 + w2[:, 0:128]) - scores[:, 0:128]
    bi = jnp.broadcast_to(lanes, v.shape)
    for c in range(1, k // 128):
        sl = slice(c * 128, (c + 1) * 128)
        d = (z2 + w2[:, sl]) - scores[:, sl]
        cmp = d < v
        v = jnp.where(cmp, d, v)
        bi = jnp.where(cmp, lanes + float(c * 128), bi)
    m = jnp.min(v, axis=1, keepdims=True)
    idx_f = jnp.min(jnp.where(v == m, bi, float(k)), axis=1, keepdims=True)
    idx_ref[...] = idx_f.astype(jnp.int32).reshape(1, blk // 128, 128)
    loss_ref[...] = loss_ref[...] + jnp.full((1, 1), jnp.sum(m), jnp.float32)


def _make_sc_gather_hist(n_tok, n_emb, dim, nc, ns):
    nw = nc * ns
    bpw = n_tok // nw
    nch = bpw // CHUNK
    half = nch // 2

    mesh = plsc.VectorSubcoreMesh(core_axis_name="c", subcore_axis_name="s")

    @functools.partial(
        pl.kernel,
        out_type=[
            jax.ShapeDtypeStruct((n_tok, dim), jnp.float32),
            jax.ShapeDtypeStruct((nc, n_emb, HW), jnp.float32),
        ],
        mesh=mesh,
        scratch_types=[
            pltpu.VMEM((nch, CHUNK), jnp.int32),
            pltpu.VMEM((half * CHUNK, dim), jnp.float32),
            pltpu.VMEM((CHUNK, HW), jnp.float32),
            pltpu.VMEM_SHARED((n_emb, HW), jnp.float32),
            pltpu.SemaphoreType.DMA,
            pltpu.SemaphoreType.DMA,
        ],
        compiler_params=pltpu.CompilerParams(use_tc_tiling_on_sc=False),
    )
    def sc_kernel(w_hbm, idx_hbm, ones_hbm, zeros_hbm, q_hbm, cnt_hbm,
                  idx_v, rows_v, ones_v, counts_v, sem_g, sem_h):
        c = lax.axis_index("c")
        s = lax.axis_index("s")
        wid = s * nc + c
        rps = n_emb // ns
        pltpu.sync_copy(idx_hbm.at[pl.ds(wid * nch, nch)], idx_v)
        pltpu.sync_copy(zeros_hbm.at[pl.ds(s * rps, rps)],
                        counts_v.at[pl.ds(s * rps, rps)])
        pltpu.sync_copy(ones_hbm, ones_v)
        plsc.subcore_barrier()
        hists = [
            pltpu.async_copy(ones_v, counts_v.at[idx_v.at[j]], sem_h, add=True)
            for j in range(nch)
        ]
        for h2 in range(2):
            gathers = [
                pltpu.async_copy(w_hbm.at[idx_v.at[h2 * half + j]],
                                 rows_v.at[pl.ds(j * CHUNK, CHUNK)], sem_g)
                for j in range(half)
            ]
            for g in gathers:
                g.wait()
            pltpu.sync_copy(
                rows_v, q_hbm.at[pl.ds(wid * bpw + h2 * half * CHUNK,
                                       half * CHUNK)])
        for h in hists:
            h.wait()
        plsc.subcore_barrier()

        @pl.when(s == 0)
        def _export():
            pltpu.sync_copy(counts_v, cnt_hbm.at[c])

    return sc_kernel


def _finalize_kernel(cnt_ref, loss_ref, loss_out, perp_out, *, n_tok, dim):
    counts = jnp.sum(cnt_ref[...], axis=0, keepdims=True)
    probs = counts * (1.0 / n_tok)
    ent = -jnp.sum(probs * jnp.log(probs + 1e-10))
    perp_out[...] = jnp.full((1, 1), jnp.exp(ent), dtype=jnp.float32)
    loss_out[...] = jnp.full(
        (1, 1), 0.25 * loss_ref[0, 0] / (n_tok * dim), dtype=jnp.float32)


def _vq_forward(z_e, w, blk, interpret=False):
    n_tok, dim = z_e.shape
    n_emb = w.shape[0]
    nb = n_tok // blk
    idx, loss_sum = pl.pallas_call(
        _dist_argmin_kernel,
        grid=(nb,),
        in_specs=[
            pl.BlockSpec((blk, dim), lambda i: (i, 0)),
            pl.BlockSpec((dim, n_emb), lambda i: (0, 0)),
        ],
        out_specs=[
            pl.BlockSpec((1, blk // 128, 128), lambda i: (i, 0, 0)),
            pl.BlockSpec((1, 1), lambda i: (0, 0)),
        ],
        out_shape=[
            jax.ShapeDtypeStruct((nb, blk // 128, 128), jnp.int32),
            jax.ShapeDtypeStruct((1, 1), jnp.float32),
        ],
        scratch_shapes=[pltpu.VMEM((1, n_emb), jnp.float32)],
        compiler_params=pltpu.CompilerParams(
            dimension_semantics=("arbitrary",),
        ),
        interpret=interpret,
    )(2.0 * z_e, w.T)

    info = plsc.get_sparse_core_info()
    nc, ns = info.num_cores, info.num_subcores
    sc = _make_sc_gather_hist(n_tok, n_emb, dim, nc, ns)
    idx_grid = idx.reshape(n_tok // CHUNK, CHUNK)
    ones = jnp.zeros((CHUNK, HW), jnp.float32).at[:, 0].set(1.0)
    zeros = jnp.zeros((n_emb, HW), jnp.float32)
    q, cnt = sc(w, idx_grid, ones, zeros)
    cnt = cnt[:, :, 0]

    loss, perp = pl.pallas_call(
        functools.partial(_finalize_kernel, n_tok=n_tok, dim=dim),
        out_shape=[
            jax.ShapeDtypeStruct((1, 1), jnp.float32),
            jax.ShapeDtypeStruct((1, 1), jnp.float32),
        ],
        interpret=interpret,
    )(cnt, loss_sum)
    return loss.reshape(()), q, perp.reshape(())


def kernel(z_e, embedding_weight):
    loss, q, perp = _vq_forward(z_e, embedding_weight, BLK)
    return (loss, q, perp)


# speedup vs baseline: 1.4371x; 1.0534x over previous
import functools

import jax
import jax.numpy as jnp
from jax import lax
from jax.experimental import pallas as pl
from jax.experimental.pallas import tpu as pltpu
from jax.experimental.pallas import tpu_sc as plsc

BLK = 1024
CHUNK = 128
HW = 8


def _dist_argmin_kernel(z_ref, wt_ref, idx_ref, loss_ref, w2_s):
    i = pl.program_id(0)
    blk, k = z_ref.shape[0], wt_ref.shape[1]

    @pl.when(i == 0)
    def _init():
        wt0 = wt_ref[...]
        w2_s[...] = jnp.sum(wt0 * wt0, axis=0, keepdims=True)
        loss_ref[...] = jnp.zeros_like(loss_ref)

    z = z_ref[...] * 2.0
    z2 = jnp.sum(z * z, axis=1, keepdims=True) * 0.25
    scores = jnp.dot(z, wt_ref[...], preferred_element_type=jnp.float32)
    w2 = w2_s[...]
    lanes = lax.broadcasted_iota(jnp.int32, (1, 128), 1).astype(jnp.float32)
    v = (z2 + w2[:, 0:128]) - scores[:, 0:128]
    bi = jnp.broadcast_to(lanes, v.shape)
    for c in range(1, k // 128):
        sl = slice(c * 128, (c + 1) * 128)
        d = (z2 + w2[:, sl]) - scores[:, sl]
        cmp = d < v
        v = jnp.where(cmp, d, v)
        bi = jnp.where(cmp, lanes + float(c * 128), bi)
    m = jnp.min(v, axis=1, keepdims=True)
    idx_f = jnp.min(jnp.where(v == m, bi, float(k)), axis=1, keepdims=True)
    idx_ref[...] = idx_f.astype(jnp.int32).reshape(blk // 128, 128)
    loss_ref[...] = loss_ref[...] + jnp.full((1, 1), jnp.sum(m), jnp.float32)


def _make_sc_gather_hist(n_tok, n_emb, dim, nc, ns):
    nw = nc * ns
    bpw = n_tok // nw
    nch = bpw // CHUNK
    half = nch // 2

    mesh = plsc.VectorSubcoreMesh(core_axis_name="c", subcore_axis_name="s")

    @functools.partial(
        pl.kernel,
        out_type=[
            jax.ShapeDtypeStruct((n_tok, dim), jnp.float32),
            jax.ShapeDtypeStruct((nc, n_emb, HW), jnp.float32),
        ],
        mesh=mesh,
        scratch_types=[
            pltpu.VMEM((nch, CHUNK), jnp.int32),
            pltpu.VMEM((half * CHUNK, dim), jnp.float32),
            pltpu.VMEM((CHUNK, HW), jnp.float32),
            pltpu.VMEM_SHARED((n_emb, HW), jnp.float32),
            pltpu.SemaphoreType.DMA,
            pltpu.SemaphoreType.DMA,
        ],
        compiler_params=pltpu.CompilerParams(use_tc_tiling_on_sc=False),
    )
    def sc_kernel(w_hbm, idx_hbm, ones_hbm, zeros_hbm, q_hbm, cnt_hbm,
                  idx_v, rows_v, ones_v, counts_v, sem_g, sem_h):
        c = lax.axis_index("c")
        s = lax.axis_index("s")
        wid = s * nc + c
        rps = n_emb // ns
        pltpu.sync_copy(idx_hbm.at[pl.ds(wid * nch, nch)], idx_v)
        pltpu.sync_copy(zeros_hbm.at[pl.ds(s * rps, rps)],
                        counts_v.at[pl.ds(s * rps, rps)])
        pltpu.sync_copy(ones_hbm, ones_v)
        plsc.subcore_barrier()
        hists = [
            pltpu.async_copy(ones_v, counts_v.at[idx_v.at[j]], sem_h, add=True)
            for j in range(nch)
        ]
        for h2 in range(2):
            gathers = [
                pltpu.async_copy(w_hbm.at[idx_v.at[h2 * half + j]],
                                 rows_v.at[pl.ds(j * CHUNK, CHUNK)], sem_g)
                for j in range(half)
            ]
            for g in gathers:
                g.wait()
            pltpu.sync_copy(
                rows_v, q_hbm.at[pl.ds(wid * bpw + h2 * half * CHUNK,
                                       half * CHUNK)])
        for h in hists:
            h.wait()
        plsc.subcore_barrier()

        @pl.when(s == 0)
        def _export():
            pltpu.sync_copy(counts_v, cnt_hbm.at[c])

    return sc_kernel


def _finalize_kernel(cnt_ref, loss_ref, loss_out, perp_out, *, n_tok, dim):
    counts = jnp.sum(cnt_ref[...], axis=0, keepdims=True)
    probs = counts * (1.0 / n_tok)
    ent = -jnp.sum(probs * jnp.log(probs + 1e-10))
    perp_out[...] = jnp.full((1, 1), jnp.exp(ent), dtype=jnp.float32)
    loss_out[...] = jnp.full(
        (1, 1), 0.25 * loss_ref[0, 0] / (n_tok * dim), dtype=jnp.float32)


def _vq_forward(z_e, w, blk, interpret=False):
    n_tok, dim = z_e.shape
    n_emb = w.shape[0]
    nb = n_tok // blk
    idx, loss_sum = pl.pallas_call(
        _dist_argmin_kernel,
        grid=(nb,),
        in_specs=[
            pl.BlockSpec((blk, dim), lambda i: (i, 0)),
            pl.BlockSpec((dim, n_emb), lambda i: (0, 0)),
        ],
        out_specs=[
            pl.BlockSpec((blk // 128, 128), lambda i: (i, 0)),
            pl.BlockSpec((1, 1), lambda i: (0, 0)),
        ],
        out_shape=[
            jax.ShapeDtypeStruct((n_tok // 128, 128), jnp.int32),
            jax.ShapeDtypeStruct((1, 1), jnp.float32),
        ],
        scratch_shapes=[pltpu.VMEM((1, n_emb), jnp.float32)],
        compiler_params=pltpu.CompilerParams(
            dimension_semantics=("arbitrary",),
        ),
        interpret=interpret,
    )(z_e, w.T)

    info = plsc.get_sparse_core_info()
    nc, ns = info.num_cores, info.num_subcores
    sc = _make_sc_gather_hist(n_tok, n_emb, dim, nc, ns)
    idx_grid = idx
    ones = jnp.zeros((CHUNK, HW), jnp.float32).at[:, 0].set(1.0)
    zeros = jnp.zeros((n_emb, HW), jnp.float32)
    q, cnt = sc(w, idx_grid, ones, zeros)
    cnt = cnt[:, :, 0]

    loss, perp = pl.pallas_call(
        functools.partial(_finalize_kernel, n_tok=n_tok, dim=dim),
        out_shape=[
            jax.ShapeDtypeStruct((1, 1), jnp.float32),
            jax.ShapeDtypeStruct((1, 1), jnp.float32),
        ],
        interpret=interpret,
    )(cnt, loss_sum)
    return loss.reshape(()), q, perp.reshape(())


def kernel(z_e, embedding_weight):
    loss, q, perp = _vq_forward(z_e, embedding_weight, BLK)
    return (loss, q, perp)


# speedup vs baseline: 1.4504x; 1.0093x over previous
import functools

import jax
import jax.numpy as jnp
from jax import lax
from jax.experimental import pallas as pl
from jax.experimental.pallas import tpu as pltpu
from jax.experimental.pallas import tpu_sc as plsc

BLK = 1024
CHUNK = 128
HW = 8


def _dist_argmin_kernel(z_ref, wt_ref, idx_ref, loss_ref, w2_s):
    i = pl.program_id(0)
    blk, k = z_ref.shape[0], wt_ref.shape[1]

    @pl.when(i == 0)
    def _init():
        wt0 = wt_ref[...]
        w2_s[...] = jnp.sum(wt0 * wt0, axis=0, keepdims=True)
        loss_ref[...] = jnp.zeros_like(loss_ref)

    z = z_ref[...] * 2.0
    z2 = jnp.sum(z * z, axis=1, keepdims=True) * 0.25
    scores = jnp.dot(z, wt_ref[...], preferred_element_type=jnp.float32)
    w2 = w2_s[...]
    lanes = lax.broadcasted_iota(jnp.int32, (1, 128), 1).astype(jnp.float32)
    v = (z2 + w2[:, 0:128]) - scores[:, 0:128]
    bi = jnp.broadcast_to(lanes, v.shape)
    for c in range(1, k // 128):
        sl = slice(c * 128, (c + 1) * 128)
        d = (z2 + w2[:, sl]) - scores[:, sl]
        cmp = d < v
        v = jnp.where(cmp, d, v)
        bi = jnp.where(cmp, lanes + float(c * 128), bi)
    m = jnp.min(v, axis=1, keepdims=True)
    idx_f = jnp.min(jnp.where(v == m, bi, float(k)), axis=1, keepdims=True)
    idx_ref[...] = idx_f.astype(jnp.int32).reshape(blk // 128, 128)
    loss_ref[...] = loss_ref[...] + jnp.full((1, 1), jnp.sum(m), jnp.float32)


def _make_sc_gather_hist(n_tok, n_emb, dim, nc, ns):
    nw = nc * ns
    bpw = n_tok // nw
    nch = bpw // CHUNK
    half = nch // 2

    mesh = plsc.VectorSubcoreMesh(core_axis_name="c", subcore_axis_name="s")

    @functools.partial(
        pl.kernel,
        out_type=[
            jax.ShapeDtypeStruct((n_tok, dim), jnp.float32),
            jax.ShapeDtypeStruct((nc, n_emb, HW), jnp.float32),
        ],
        mesh=mesh,
        scratch_types=[
            pltpu.VMEM((nch, CHUNK), jnp.int32),
            pltpu.VMEM((half * CHUNK, dim), jnp.float32),
            pltpu.VMEM((CHUNK, HW), jnp.float32),
            pltpu.VMEM_SHARED((n_emb, HW), jnp.float32),
            pltpu.SemaphoreType.DMA,
            pltpu.SemaphoreType.DMA,
        ],
        compiler_params=pltpu.CompilerParams(use_tc_tiling_on_sc=False),
    )
    def sc_kernel(w_hbm, idx_hbm, ones_hbm, zeros_hbm, q_hbm, cnt_hbm,
                  idx_v, rows_v, ones_v, counts_v, sem_g, sem_h):
        c = lax.axis_index("c")
        s = lax.axis_index("s")
        wid = s * nc + c
        rps = n_emb // ns
        pltpu.sync_copy(idx_hbm.at[pl.ds(wid * nch, nch)], idx_v)
        pltpu.sync_copy(zeros_hbm.at[pl.ds(s * rps, rps)],
                        counts_v.at[pl.ds(s * rps, rps)])
        pltpu.sync_copy(ones_hbm, ones_v)
        plsc.subcore_barrier()
        hists = [
            pltpu.async_copy(ones_v, counts_v.at[idx_v.at[j]], sem_h, add=True)
            for j in range(nch)
        ]
        for h2 in range(2):
            gathers = [
                pltpu.async_copy(w_hbm.at[idx_v.at[h2 * half + j]],
                                 rows_v.at[pl.ds(j * CHUNK, CHUNK)], sem_g)
                for j in range(half)
            ]
            for g in gathers:
                g.wait()
            pltpu.sync_copy(
                rows_v, q_hbm.at[pl.ds(wid * bpw + h2 * half * CHUNK,
                                       half * CHUNK)])
        for h in hists:
            h.wait()
        plsc.subcore_barrier()

        @pl.when(s == 0)
        def _export():
            pltpu.sync_copy(counts_v, cnt_hbm.at[c])

    return sc_kernel


def _finalize_kernel(cnt_ref, loss_ref, loss_out, perp_out, *, n_tok, dim):
    counts = jnp.sum(cnt_ref[...], axis=0, keepdims=True)
    probs = counts * (1.0 / n_tok)
    ent = -jnp.sum(probs * jnp.log(probs + 1e-10))
    perp_out[...] = jnp.full((1, 1), jnp.exp(ent), dtype=jnp.float32)
    loss_out[...] = jnp.full(
        (1, 1), 0.25 * loss_ref[0, 0] / (n_tok * dim), dtype=jnp.float32)


def _vq_forward(z_e, w, blk, interpret=False):
    n_tok, dim = z_e.shape
    n_emb = w.shape[0]
    nb = n_tok // blk
    idx, loss_sum = pl.pallas_call(
        _dist_argmin_kernel,
        grid=(nb,),
        in_specs=[
            pl.BlockSpec((blk, dim), lambda i: (i, 0)),
            pl.BlockSpec((dim, n_emb), lambda i: (0, 0)),
        ],
        out_specs=[
            pl.BlockSpec((blk // 128, 128), lambda i: (i, 0)),
            pl.BlockSpec((1, 1), lambda i: (0, 0)),
        ],
        out_shape=[
            jax.ShapeDtypeStruct((n_tok // 128, 128), jnp.int32),
            jax.ShapeDtypeStruct((1, 1), jnp.float32),
        ],
        scratch_shapes=[pltpu.VMEM((1, n_emb), jnp.float32)],
        compiler_params=pltpu.CompilerParams(
            dimension_semantics=("arbitrary",),
        ),
        interpret=interpret,
    )(z_e, w.T)

    info = plsc.get_sparse_core_info()
    nc, ns = info.num_cores, info.num_subcores
    sc = _make_sc_gather_hist(n_tok, n_emb, dim, nc, ns)
    idx_grid = idx
    ones = jnp.zeros((CHUNK, HW), jnp.float32).at[:, 0].set(1.0)
    zeros = jnp.zeros((n_emb, HW), jnp.float32)
    q, cnt = sc(w, idx_grid, ones, zeros)
    cnt = cnt.reshape(cnt.shape[0], cnt.shape[1] * cnt.shape[2])

    loss, perp = pl.pallas_call(
        functools.partial(_finalize_kernel, n_tok=n_tok, dim=dim),
        out_shape=[
            jax.ShapeDtypeStruct((1, 1), jnp.float32),
            jax.ShapeDtypeStruct((1, 1), jnp.float32),
        ],
        interpret=interpret,
    )(cnt, loss_sum)
    return loss.reshape(()), q, perp.reshape(())


def kernel(z_e, embedding_weight):
    loss, q, perp = _vq_forward(z_e, embedding_weight, BLK)
    return (loss, q, perp)
